# baseline (speedup 1.0000x reference)
import jax
import jax.numpy as jnp
import numpy as np
from jax.experimental import pallas as pl
from jax.experimental.pallas import tpu as pltpu

S = 2048
D = 768
H = 8
DH = 96
DHP = 128
DP = H * DHP
E = 8
TOPK = 2
FF = 3072
T = 128
NB = 40
NR = NB * T
SB = 128
SNB = S // SB
F32 = jnp.float32
_HI = jax.lax.Precision.HIGHEST
_RSQ_ATT = np.float32(np.sqrt(np.float32(DH)))
_RSQ_ROUTER = np.float32(np.sqrt(np.float32(D)))


def _fiota(shape, dim):
    return jax.lax.broadcasted_iota(jnp.int32, shape, dim).astype(F32)


def _dot(a, b, precision=None):
    return jax.lax.dot_general(a, b, (((1,), (0,)), ((), ())),
                               precision=precision, preferred_element_type=F32)


def _dott(a, b, precision=None):
    return jax.lax.dot_general(a, b, (((0,), (0,)), ((), ())),
                               precision=precision, preferred_element_type=F32)


def _dotr(a, b):
    return jax.lax.dot_general(a, b, (((1,), (1,)), ((), ())),
                               preferred_element_type=F32)


def _ln_stats(x, g, b, stats_src=None):
    x3 = x[None]
    s3 = x3 if stats_src is None else stats_src[None]
    mu = jnp.mean(s3, axis=-1, keepdims=True)
    var = jnp.var(s3, axis=-1, keepdims=True)
    return ((x3 - mu) / jnp.sqrt(var + 1e-5) * g + b)[0]


def _softmax_stats(s4):
    m = jnp.max(s4, axis=-1, keepdims=True)
    l = jnp.sum(jnp.exp(s4 - m), axis=-1, keepdims=True)
    return m, l


def _embed_body(ids_ref, emb_ref, pos_ref, o_ref):
    o_ref[...] = emb_ref[...] + pos_ref[...]


def _embed(ids, emb, pos):
    x = pl.pallas_call(
        _embed_body,
        grid_spec=pltpu.PrefetchScalarGridSpec(
            num_scalar_prefetch=1,
            grid=(S,),
            in_specs=[
                pl.BlockSpec((1, 1, D), lambda i, ids_ref: (ids_ref[i], 0, 0)),
                pl.BlockSpec((1, 1, D), lambda i, ids_ref: (i, 0, 0)),
            ],
            out_specs=pl.BlockSpec((1, 1, D), lambda i, ids_ref: (i, 0, 0)),
        ),
        out_shape=jax.ShapeDtypeStruct((S, 1, D), F32),
    )(ids, emb.reshape(-1, 1, D), pos.reshape(S, 1, D))
    return x.reshape(S, D)


def _qkv_body(x_ref, wq_ref, wq2_ref, wk_ref, wk2_ref, wv_ref,
              cos_ref, sin_ref, q_ref, k_ref, v_ref):
    xn = x_ref[...]
    cos = cos_ref[...]
    sin = sin_ref[...]
    q_ref[...] = _dot(xn, wq_ref[0]) * cos + _dot(xn, wq2_ref[0]) * sin
    k_ref[...] = _dot(xn, wk_ref[0]) * cos + _dot(xn, wk2_ref[0]) * sin
    v_ref[...] = _dot(xn, wv_ref[0])


def _qkv(xln, wq_h, wq2_h, wk_h, wk2_h, wv_h, cosp, sinp):
    wspec = pl.BlockSpec((1, D, DHP), lambda i, h: (h, 0, 0))
    return pl.pallas_call(
        _qkv_body,
        grid=(SNB, H),
        in_specs=[
            pl.BlockSpec((SB, D), lambda i, h: (i, 0)),
            wspec, wspec, wspec, wspec, wspec,
            pl.BlockSpec((SB, DHP), lambda i, h: (i, 0)),
            pl.BlockSpec((SB, DHP), lambda i, h: (i, 0)),
        ],
        out_specs=[pl.BlockSpec((SB, DHP), lambda i, h: (i, h))] * 3,
        out_shape=[jax.ShapeDtypeStruct((S, DP), F32)] * 3,
    )(xln, wq_h, wq2_h, wk_h, wk2_h, wv_h, cosp, sinp)


def _scores_body(q_ref, k_ref, s_ref):
    s_ref[0] = _dotr(q_ref[...], k_ref[...]) / _RSQ_ATT


def _scores(q, k):
    return pl.pallas_call(
        _scores_body,
        grid=(H, SNB),
        in_specs=[
            pl.BlockSpec((SB, DHP), lambda h, i: (i, h)),
            pl.BlockSpec((S, DHP), lambda h, i: (0, h)),
        ],
        out_specs=pl.BlockSpec((1, SB, S), lambda h, i: (h, i, 0)),
        out_shape=jax.ShapeDtypeStruct((H, S, S), F32),
    )(q, k)


def _attnav_body(s_ref, m_ref, l_ref, v_ref, o_ref):
    p = jnp.exp(s_ref[0] - m_ref[0]) / l_ref[0]
    o_ref[...] = _dot(p, v_ref[...])


def _attnav(s, m, l, v):
    return pl.pallas_call(
        _attnav_body,
        grid=(H, SNB),
        in_specs=[
            pl.BlockSpec((1, SB, S), lambda h, i: (h, i, 0)),
            pl.BlockSpec((1, SB, 1), lambda h, i: (h, i, 0)),
            pl.BlockSpec((1, SB, 1), lambda h, i: (h, i, 0)),
            pl.BlockSpec((S, DHP), lambda h, i: (0, h)),
        ],
        out_specs=pl.BlockSpec((SB, DHP), lambda h, i: (i, h)),
        out_shape=jax.ShapeDtypeStruct((S, DP), F32),
    )(s, m, l, v)


def _proj_body(c_ref, wo_ref, o_ref):
    o_ref[...] = _dot(c_ref[...], wo_ref[...])


def _proj(ctx_c, wo):
    return pl.pallas_call(
        _proj_body,
        grid=(SNB,),
        in_specs=[
            pl.BlockSpec((SB, D), lambda i: (i, 0)),
            pl.BlockSpec((D, D), lambda i: (0, 0)),
        ],
        out_specs=pl.BlockSpec((SB, D), lambda i: (i, 0)),
        out_shape=jax.ShapeDtypeStruct((S, D), F32),
    )(ctx_c, wo)


def _rproj_body(x_ref, wq_ref, wk_ref, q_ref, k_ref):
    x = x_ref[...]
    q_ref[...] = _dot(x, wq_ref[...])
    k_ref[...] = _dot(x, wk_ref[...])


def _rproj(xln, wrq, wrk):
    return pl.pallas_call(
        _rproj_body,
        grid=(SNB,),
        in_specs=[
            pl.BlockSpec((SB, D), lambda i: (i, 0)),
            pl.BlockSpec((D, D), lambda i: (0, 0)),
            pl.BlockSpec((D, D), lambda i: (0, 0)),
        ],
        out_specs=[pl.BlockSpec((SB, D), lambda i: (i, 0))] * 2,
        out_shape=[jax.ShapeDtypeStruct((S, D), F32)] * 2,
    )(xln, wrq, wrk)


def _rscores_body(q_ref, k_ref, s_ref):
    s_ref[...] = _dotr(q_ref[...], k_ref[...]) / _RSQ_ROUTER


def _rscores(qr, kr):
    return pl.pallas_call(
        _rscores_body,
        grid=(SNB,),
        in_specs=[
            pl.BlockSpec((SB, D), lambda i: (i, 0)),
            pl.BlockSpec((S, D), lambda i: (0, 0)),
        ],
        out_specs=pl.BlockSpec((SB, S), lambda i: (i, 0)),
        out_shape=jax.ShapeDtypeStruct((S, S), F32),
    )(qr, kr)


def _rctx_body(s_ref, m_ref, l_ref, x_ref, wr_ref, lg_ref):
    i = pl.program_id(0)
    p = jnp.exp(s_ref[...] - m_ref[...]) / l_ref[...]
    ctx = _dot(p, x_ref[...])
    xblk = x_ref[pl.ds(i * SB, SB), :]
    lg_ref[...] = _dot(xblk + ctx, wr_ref[...])


def _rctx(s, m, l, xln, wr):
    return pl.pallas_call(
        _rctx_body,
        grid=(SNB,),
        in_specs=[
            pl.BlockSpec((SB, S), lambda i: (i, 0)),
            pl.BlockSpec((SB, 1), lambda i: (i, 0)),
            pl.BlockSpec((SB, 1), lambda i: (i, 0)),
            pl.BlockSpec((S, D), lambda i: (0, 0)),
            pl.BlockSpec((D, E), lambda i: (0, 0)),
        ],
        out_specs=pl.BlockSpec((SB, E), lambda i: (i, 0)),
        out_shape=jax.ShapeDtypeStruct((S, E), F32),
    )(s, m, l, xln, wr)


def _route_body(lg_ref, mask_ref, loss_ref, p0_ref, p1_ref, g0_ref, g1_ref,
                be_ref, live_ref):
    lg = lg_ref[...]
    ie = _fiota((S, E), 1)
    v0 = jnp.max(lg, axis=1, keepdims=True)
    i0 = jnp.min(jnp.where(lg == v0, ie, float(E)), axis=1, keepdims=True)
    oh0 = (ie == i0).astype(F32)
    lg2 = jnp.where(oh0 > 0, -jnp.inf, lg)
    v1 = jnp.max(lg2, axis=1, keepdims=True)
    i1 = jnp.min(jnp.where(lg2 == v1, ie, float(E)), axis=1, keepdims=True)
    oh1 = (ie == i1).astype(F32)
    d = jnp.exp(v1 - v0)
    den = 1.0 + d
    g0 = 1.0 / den
    g1 = d / den
    mask_ref[...] = oh0 * g0 + oh1 * g1
    pm = jnp.exp(lg - v0)
    pm = pm / jnp.sum(pm, axis=1, keepdims=True)
    pmm = jnp.sum(pm, axis=0, keepdims=True) * (1.0 / S)
    ohb = oh0 + oh1
    cnt = jnp.sum(ohb, axis=0, keepdims=True)
    frac = cnt * (1.0 / (S * TOPK))
    loss_ref[...] = jnp.sum(frac * pmm, axis=1, keepdims=True) * float(E)
    tri = (_fiota((SB, SB), 0) > _fiota((SB, SB), 1)).astype(F32)
    carry = jnp.zeros((1, E), F32)
    chunks = []
    for j in range(S // SB):
        ch = ohb[j * SB:(j + 1) * SB, :]
        chunks.append(_dot(tri, ch) + carry)
        carry = carry + jnp.sum(ch, axis=0, keepdims=True)
    C = jnp.concatenate(chunks, axis=0)
    pe = jnp.ceil(cnt * (1.0 / T)) * float(T)
    mlt = (_fiota((E, E), 0) < _fiota((E, E), 1)).astype(F32)
    off = _dot(pe, mlt)
    pos_base = C + off
    p0_ref[...] = jnp.sum(pos_base * oh0, axis=1, keepdims=True)
    p1_ref[...] = jnp.sum(pos_base * oh1, axis=1, keepdims=True)
    g0_ref[...] = g0
    g1_ref[...] = g1
    nb_live = jnp.sum(pe, axis=1, keepdims=True) * (1.0 / T)
    bi = _fiota((1, NB), 1)
    bcl = jnp.minimum(bi, nb_live - 1.0)
    cum_blocks = (off + pe) * (1.0 / T)
    be = jnp.zeros((1, NB), F32)
    for e in range(E):
        be = be + (bcl >= cum_blocks[:, e:e + 1]).astype(F32)
    be_ref[...] = be.astype(jnp.int32)
    live_ref[...] = (bi < nb_live).astype(jnp.int32)


def _route(logits):
    return pl.pallas_call(
        _route_body,
        grid=(1,),
        in_specs=[pl.BlockSpec((S, E), lambda i: (0, 0))],
        out_specs=[
            pl.BlockSpec((S, E), lambda i: (0, 0)),
            pl.BlockSpec((1, 1), lambda i: (0, 0)),
            pl.BlockSpec((S, 1), lambda i: (0, 0)),
            pl.BlockSpec((S, 1), lambda i: (0, 0)),
            pl.BlockSpec((S, 1), lambda i: (0, 0)),
            pl.BlockSpec((S, 1), lambda i: (0, 0)),
            pl.BlockSpec((1, NB), lambda i: (0, 0)),
            pl.BlockSpec((1, NB), lambda i: (0, 0)),
        ],
        out_shape=[
            jax.ShapeDtypeStruct((S, E), F32),
            jax.ShapeDtypeStruct((1, 1), F32),
            jax.ShapeDtypeStruct((S, 1), F32),
            jax.ShapeDtypeStruct((S, 1), F32),
            jax.ShapeDtypeStruct((S, 1), F32),
            jax.ShapeDtypeStruct((S, 1), F32),
            jax.ShapeDtypeStruct((1, NB), jnp.int32),
            jax.ShapeDtypeStruct((1, NB), jnp.int32),
        ],
    )(logits)


def _ffn_body(be_ref, live_ref, p0_ref, p1_ref, g0_ref, g1_ref, x_ref,
              w1_ref, b1_ref, w2_ref, b2_ref, o_ref):
    b = pl.program_id(0)
    base = jnp.float32(b * T)

    @pl.when(live_ref[b] == 1)
    def _():
        it = _fiota((S, T), 1) + base
        oh0 = (p0_ref[...] == it).astype(F32)
        oh1 = (p1_ref[...] == it).astype(F32)
        xg = _dott(oh0 + oh1, x_ref[...], precision=_HI)
        h = jnp.maximum(_dot(xg, w1_ref[0]) + b1_ref[0], 0.0)
        y = _dot(h, w2_ref[0]) + b2_ref[0]
        grow = (_dott(oh0, g0_ref[...], precision=_HI) +
                _dott(oh1, g1_ref[...], precision=_HI))
        o_ref[...] = y * grow

    @pl.when(live_ref[b] == 0)
    def _():
        o_ref[...] = jnp.zeros((T, D), F32)


def _ffn(be, live, p0, p1, g0, g1, xln, w1, b1, w2, b2):
    return pl.pallas_call(
        _ffn_body,
        grid_spec=pltpu.PrefetchScalarGridSpec(
            num_scalar_prefetch=2,
            grid=(NB,),
            in_specs=[
                pl.BlockSpec((S, 1), lambda b, be, lv: (0, 0)),
                pl.BlockSpec((S, 1), lambda b, be, lv: (0, 0)),
                pl.BlockSpec((S, 1), lambda b, be, lv: (0, 0)),
                pl.BlockSpec((S, 1), lambda b, be, lv: (0, 0)),
                pl.BlockSpec((S, D), lambda b, be, lv: (0, 0)),
                pl.BlockSpec((1, D, FF), lambda b, be, lv: (be[b], 0, 0)),
                pl.BlockSpec((1, 1, FF), lambda b, be, lv: (be[b], 0, 0)),
                pl.BlockSpec((1, FF, D), lambda b, be, lv: (be[b], 0, 0)),
                pl.BlockSpec((1, 1, D), lambda b, be, lv: (be[b], 0, 0)),
            ],
            out_specs=pl.BlockSpec((T, D), lambda b, be, lv: (b, 0)),
        ),
        out_shape=jax.ShapeDtypeStruct((NR, D), F32),
    )(be, live, p0, p1, g0, g1, xln, w1, b1.reshape(E, 1, FF), w2,
      b2.reshape(E, 1, D))


def _combine_body(yg_ref, p0_ref, p1_ref, o_ref):
    inr = _fiota((SB, NR), 1)
    oh = (p0_ref[...] == inr).astype(F32) + (p1_ref[...] == inr).astype(F32)
    o_ref[...] = _dot(oh, yg_ref[...], precision=_HI)


def _combine(yg, p0, p1):
    return pl.pallas_call(
        _combine_body,
        grid=(SNB,),
        in_specs=[
            pl.BlockSpec((NR, D), lambda i: (0, 0)),
            pl.BlockSpec((SB, 1), lambda i: (i, 0)),
            pl.BlockSpec((SB, 1), lambda i: (i, 0)),
        ],
        out_specs=pl.BlockSpec((SB, D), lambda i: (i, 0)),
        out_shape=jax.ShapeDtypeStruct((S, D), F32),
    )(yg, p0, p1)


def _final_body(o_ref, wc_ref, bc_ref, fv_ref, lg_ref):
    fv = jnp.sum(o_ref[...], axis=0, keepdims=True) * (1.0 / S)
    fv_ref[...] = fv
    lg_ref[...] = _dot(fv, wc_ref[...]) + bc_ref[...]


def _final(o2, wc, bc):
    return pl.pallas_call(
        _final_body,
        grid=(1,),
        in_specs=[
            pl.BlockSpec((S, D), lambda i: (0, 0)),
            pl.BlockSpec((D, 10), lambda i: (0, 0)),
            pl.BlockSpec((1, 10), lambda i: (0, 0)),
        ],
        out_specs=[
            pl.BlockSpec((1, D), lambda i: (0, 0)),
            pl.BlockSpec((1, 10), lambda i: (0, 0)),
        ],
        out_shape=[
            jax.ShapeDtypeStruct((1, D), F32),
            jax.ShapeDtypeStruct((1, 10), F32),
        ],
    )(o2, wc, bc.reshape(1, 10))


def _moe_layer(xln, p):
    qr, kr = _rproj(xln, p['Wrq'], p['Wrk'])
    sr = _rscores(qr, kr)
    m, l = _softmax_stats(sr[None])
    logits = _rctx(sr, m[0], l[0], xln, p['Wr'])
    mask, loss, p0, p1, g0, g1, be, live = _route(logits)
    yg = _ffn(be.reshape(NB), live.reshape(NB), p0, p1, g0, g1,
              xln, p['W1'], p['b1'], p['W2'], p['b2'])
    o = _combine(yg, p0, p1)
    return o, mask, loss


def kernel(input_ids, attention_mask, params):
    del attention_mask
    p = params
    ids = input_ids.reshape(S).astype(jnp.int32)

    half = DH // 2
    inv_freq = 1.0 / (10000.0 ** (jnp.arange(half, dtype=F32) / half))
    t = jnp.arange(S, dtype=F32)
    fr = t[:, None] * inv_freq[None, :]
    cos1 = jnp.cos(fr)
    sin1 = jnp.sin(fr)
    cosp = jnp.pad(jnp.concatenate([cos1, cos1], axis=1),
                   ((0, 0), (0, DHP - DH)))
    sinp = jnp.pad(jnp.concatenate([-sin1, sin1], axis=1),
                   ((0, 0), (0, DHP - DH)))

    def per_head(w):
        wh = w.reshape(D, H, DH).transpose(1, 0, 2)
        return jnp.pad(wh, ((0, 0), (0, 0), (0, DHP - DH)))

    def shuf(wh):
        return jnp.concatenate(
            [wh[:, :, half:DH], wh[:, :, :half], wh[:, :, DH:]], axis=2)

    wq_h = per_head(p['Wq'])
    wk_h = per_head(p['Wk'])
    wv_h = per_head(p['Wv'])
    wq2_h = shuf(wq_h)
    wk2_h = shuf(wk_h)

    x0 = _embed(ids, p['emb'], p['pos'])
    x0_clone = (jnp.take(p['emb'], input_ids, axis=0) + p['pos'][None])[0]
    x0ln = _ln_stats(x0, p['ln1_g'], p['ln1_b'], stats_src=x0_clone)
    q, k, v = _qkv(x0ln, wq_h, wq2_h, wk_h, wk2_h, wv_h, cosp, sinp)
    s = _scores(q, k)
    m, l = _softmax_stats(s[None])
    ctx = _attnav(s, m[0], l[0], v)
    ctx_c = ctx.reshape(S, H, DHP)[:, :, :DH].reshape(S, D)
    x1 = x0 + _proj(ctx_c, p['Wo'])
    x1ln = _ln_stats(x1, p['ln2_g'], p['ln2_b'])

    o1, m1, l1 = _moe_layer(x1ln, p['moe1'])
    o1ln = _ln_stats(o1, p['ln3_g'], p['ln3_b'])
    o2, m2, l2 = _moe_layer(o1ln, p['moe2'])

    fv, lg = _final(o2, p['Wc'], p['bc'])

    total_loss = (l1[0, 0] + l2[0, 0]).astype(F32)
    return (lg, fv, total_loss,
            m1.reshape(1, S, E), m2.reshape(1, S, E))
